# baseline (speedup 1.0000x reference)
import functools

import jax
import jax.numpy as jnp
from jax import lax
from jax.experimental import pallas as pl
from jax.experimental.pallas import tpu as pltpu
from jax.experimental.pallas import tpu_sc as plsc

C = 10
N = 16384
D = 64
L = 16
NW = 32
RPW = N // NW
NCH = D // L
NG = RPW // L
CD = C * D
CL = C * L


def _sqrt16(a):
    i = lax.bitcast_convert_type(a, jnp.int32)
    z = lax.bitcast_convert_type(jnp.int32(0x5F3759DF) - (i >> 1), jnp.float32)
    for _ in range(3):
        z = z * (1.5 - (0.5 * a) * z * z)
    return a * z


def _sc_body(
    pred_hbm, tgt_hbm, centp_hbm,
    partA_hbm, partB_hbm,
    pred_v, tgt_v, centp_v, acc_v, acc2_v, accvl_v,
):
    wid = lax.axis_index("s") * 2 + lax.axis_index("c")
    base = wid * RPW

    pltpu.sync_copy(pred_hbm.at[pl.ds(base * D, RPW * D)], pred_v)
    pltpu.sync_copy(tgt_hbm.at[pl.ds(base, RPW)], tgt_v)
    pltpu.sync_copy(centp_hbm, centp_v)

    zv = jnp.zeros((L,), jnp.float32)
    for q in range(CD // L):
        acc_v[pl.ds(q * L, L)] = zv
        acc2_v[pl.ds(q * L, L)] = zv
    for q in range(CL // L):
        accvl_v[pl.ds(q * L, L)] = zv

    lane_iota = lax.iota(jnp.int32, L)

    def grp_body(g, carry):
        tvec = tgt_v[pl.ds(g * L, L)]
        svec = jnp.zeros((L,), jnp.float32)
        for lane in range(L):
            cls = tvec[lane]
            rb = (g * L + lane) * D
            cb = cls * D
            tgt_acc = acc_v if lane % 2 == 0 else acc2_v
            s2a = jnp.zeros((L,), jnp.float32)
            s2b = jnp.zeros((L,), jnp.float32)
            for k in range(NCH):
                x = pred_v[pl.ds(rb + k * L, L)]
                diff = centp_v[pl.ds(cb + k * L, L)] - x
                if k % 2 == 0:
                    s2a = s2a + diff * diff
                else:
                    s2b = s2b + diff * diff
                plsc.addupdate(tgt_acc.at[pl.ds(cb + k * L, L)], x)
            svec = jnp.where(lane_iota == lane, jnp.sum(s2a + s2b), svec)
        nrm = _sqrt16(svec)
        plsc.addupdate_scatter(accvl_v, [tvec * L + lane_iota], nrm)
        return carry

    lax.fori_loop(0, NG, grp_body, 0)

    for q in range(CD // L):
        plsc.addupdate(acc_v.at[pl.ds(q * L, L)], acc2_v[pl.ds(q * L, L)])

    pltpu.sync_copy(acc_v, partA_hbm.at[pl.ds(wid * CD, CD)])
    pltpu.sync_copy(accvl_v, partB_hbm.at[pl.ds(wid * CL, CL)])


def _tc_tail_body(partA_ref, partB_ref, cent_ref, count_ref, dist_ref, w_ref,
                  out_ref):
    q = 1.0 / count_ref[...][:, 0]
    c2 = cent_ref[...] + q[:, None] * jnp.sum(partA_ref[...], axis=0)
    nrmsum = q * jnp.sum(partB_ref[...], axis=(0, 2))
    s = jnp.sqrt(dist_ref[...][:, 0] + nrmsum) * q
    gram = lax.dot_general(c2, c2, (((1,), (1,)), ((), ())),
                           preferred_element_type=jnp.float32)
    cs = jnp.sum(c2 * c2, axis=1)
    d2 = cs[:, None] + cs[None, :] - 2.0 * gram
    eye = (lax.broadcasted_iota(jnp.int32, (C, C), 0)
           == lax.broadcasted_iota(jnp.int32, (C, C), 1))
    m = jnp.sqrt(jnp.where(eye, 1.0, jnp.maximum(d2, 0.0)))
    pair = w_ref[...] * (s[:, None] + s[None, :]) / m
    total = jnp.sum(jnp.where(eye, 0.0, pair))
    loss = total / C * (C - 1) + jnp.sum(jnp.abs(c2)) / 1000000.0
    out_ref[...] = jnp.full((1, 1), 0.0) + loss


@jax.jit
def _db_loss(pred, tgt, centp, cent2d, count2d, dist2d, w2d):
    mesh = plsc.VectorSubcoreMesh(core_axis_name="c", subcore_axis_name="s")
    params = pltpu.CompilerParams(needs_layout_passes=False)

    sc = functools.partial(
        pl.kernel,
        out_type=[
            jax.ShapeDtypeStruct((NW * CD,), jnp.float32),
            jax.ShapeDtypeStruct((NW * CL,), jnp.float32),
        ],
        mesh=mesh,
        compiler_params=params,
        scratch_types=[
            pltpu.VMEM((RPW * D,), jnp.float32),
            pltpu.VMEM((RPW,), jnp.int32),
            pltpu.VMEM((CD,), jnp.float32),
            pltpu.VMEM((CD,), jnp.float32),
            pltpu.VMEM((CD,), jnp.float32),
            pltpu.VMEM((CL,), jnp.float32),
        ],
    )(_sc_body)
    partA, partB = sc(pred, tgt, centp)

    out = pl.pallas_call(
        _tc_tail_body,
        in_specs=[
            pl.BlockSpec((NW, C, D), lambda: (0, 0, 0)),
            pl.BlockSpec((NW, C, L), lambda: (0, 0, 0)),
            pl.BlockSpec((C, D), lambda: (0, 0)),
            pl.BlockSpec((C, 1), lambda: (0, 0)),
            pl.BlockSpec((C, 1), lambda: (0, 0)),
            pl.BlockSpec((C, C), lambda: (0, 0)),
        ],
        out_specs=pl.BlockSpec((1, 1), lambda: (0, 0)),
        out_shape=jax.ShapeDtypeStruct((1, 1), jnp.float32),
    )(partA.reshape(NW, C, D), partB.reshape(NW, C, L), cent2d,
      count2d, dist2d, w2d)
    return out


def kernel(predicted, centroids, count, distances, class_weights_matrix, target, epoch):
    centp = (centroids * count).reshape(CD)
    out = _db_loss(predicted.reshape(N * D), target.astype(jnp.int32),
                   centp, centroids, count, distances, class_weights_matrix)
    return out.reshape(1)


# speedup vs baseline: 1.0088x; 1.0088x over previous
import functools

import jax
import jax.numpy as jnp
from jax import lax
from jax.experimental import pallas as pl
from jax.experimental.pallas import tpu as pltpu
from jax.experimental.pallas import tpu_sc as plsc

C = 10
N = 16384
D = 64
L = 16
NW = 32
RPW = N // NW
NCH = D // L
NG = RPW // L
CD = C * D
CL = C * L


def _sqrt16(a):
    i = lax.bitcast_convert_type(a, jnp.int32)
    z = lax.bitcast_convert_type(jnp.int32(0x5F3759DF) - (i >> 1), jnp.float32)
    for _ in range(3):
        z = z * (1.5 - (0.5 * a) * z * z)
    return a * z


def _sc_body(
    pred_hbm, tgt_hbm, centp_hbm,
    partA_hbm, partB_hbm,
    pred_v, tgt_v, centp_v, acc_v, accvl_v,
):
    wid = lax.axis_index("s") * 2 + lax.axis_index("c")
    base = wid * RPW

    pltpu.sync_copy(pred_hbm.at[pl.ds(base * D, RPW * D)], pred_v)
    pltpu.sync_copy(tgt_hbm.at[pl.ds(base, RPW)], tgt_v)
    pltpu.sync_copy(centp_hbm, centp_v)

    zv = jnp.zeros((L,), jnp.float32)
    for q in range(CD // L):
        acc_v[pl.ds(q * L, L)] = zv
    for q in range(CL // L):
        accvl_v[pl.ds(q * L, L)] = zv

    lane_iota = lax.iota(jnp.int32, L)

    def grp_body(g, carry):
        tvec = tgt_v[pl.ds(g * L, L)]
        svec = jnp.zeros((L,), jnp.float32)
        for lane in range(L):
            cls = tvec[lane]
            rb = (g * L + lane) * D
            cb = cls * D
            s2 = jnp.zeros((L,), jnp.float32)
            for k in range(NCH):
                x = pred_v[pl.ds(rb + k * L, L)]
                diff = centp_v[pl.ds(cb + k * L, L)] - x
                s2 = s2 + diff * diff
                plsc.addupdate(acc_v.at[pl.ds(cb + k * L, L)], x)
            svec = jnp.where(lane_iota == lane, jnp.sum(s2), svec)
        nrm = _sqrt16(svec)
        plsc.addupdate_scatter(accvl_v, [tvec * L + lane_iota], nrm)
        return carry

    lax.fori_loop(0, NG, grp_body, 0)

    pltpu.sync_copy(acc_v, partA_hbm.at[pl.ds(wid * CD, CD)])
    pltpu.sync_copy(accvl_v, partB_hbm.at[pl.ds(wid * CL, CL)])


def _tc_tail_body(partA_ref, partB_ref, cent_ref, count_ref, dist_ref, w_ref,
                  out_ref):
    q = 1.0 / count_ref[...][:, 0]
    c2 = cent_ref[...] + q[:, None] * jnp.sum(partA_ref[...], axis=0)
    nrmsum = q * jnp.sum(partB_ref[...], axis=(0, 2))
    s = jnp.sqrt(dist_ref[...][:, 0] + nrmsum) * q
    gram = lax.dot_general(c2, c2, (((1,), (1,)), ((), ())),
                           preferred_element_type=jnp.float32)
    cs = jnp.sum(c2 * c2, axis=1)
    d2 = cs[:, None] + cs[None, :] - 2.0 * gram
    eye = (lax.broadcasted_iota(jnp.int32, (C, C), 0)
           == lax.broadcasted_iota(jnp.int32, (C, C), 1))
    m = jnp.sqrt(jnp.where(eye, 1.0, jnp.maximum(d2, 0.0)))
    pair = w_ref[...] * (s[:, None] + s[None, :]) / m
    total = jnp.sum(jnp.where(eye, 0.0, pair))
    loss = total / C * (C - 1) + jnp.sum(jnp.abs(c2)) / 1000000.0
    out_ref[...] = jnp.full((1, 1), 0.0) + loss


@jax.jit
def _db_loss(pred, tgt, centp, cent2d, count2d, dist2d, w2d):
    mesh = plsc.VectorSubcoreMesh(core_axis_name="c", subcore_axis_name="s")
    params = pltpu.CompilerParams(needs_layout_passes=False)

    sc = functools.partial(
        pl.kernel,
        out_type=[
            jax.ShapeDtypeStruct((NW * CD,), jnp.float32),
            jax.ShapeDtypeStruct((NW * CL,), jnp.float32),
        ],
        mesh=mesh,
        compiler_params=params,
        scratch_types=[
            pltpu.VMEM((RPW * D,), jnp.float32),
            pltpu.VMEM((RPW,), jnp.int32),
            pltpu.VMEM((CD,), jnp.float32),
            pltpu.VMEM((CD,), jnp.float32),
            pltpu.VMEM((CL,), jnp.float32),
        ],
    )(_sc_body)
    partA, partB = sc(pred, tgt, centp)

    out = pl.pallas_call(
        _tc_tail_body,
        in_specs=[
            pl.BlockSpec((NW, C, D), lambda: (0, 0, 0)),
            pl.BlockSpec((NW, C, L), lambda: (0, 0, 0)),
            pl.BlockSpec((C, D), lambda: (0, 0)),
            pl.BlockSpec((C, 1), lambda: (0, 0)),
            pl.BlockSpec((C, 1), lambda: (0, 0)),
            pl.BlockSpec((C, C), lambda: (0, 0)),
        ],
        out_specs=pl.BlockSpec((1, 1), lambda: (0, 0)),
        out_shape=jax.ShapeDtypeStruct((1, 1), jnp.float32),
    )(partA.reshape(NW, C, D), partB.reshape(NW, C, L), cent2d,
      count2d, dist2d, w2d)
    return out


def kernel(predicted, centroids, count, distances, class_weights_matrix, target, epoch):
    centp = (centroids * count).reshape(CD)
    out = _db_loss(predicted.reshape(N * D), target.astype(jnp.int32),
                   centp, centroids, count, distances, class_weights_matrix)
    return out.reshape(1)


# speedup vs baseline: 1.0153x; 1.0065x over previous
import functools

import jax
import jax.numpy as jnp
from jax import lax
from jax.experimental import pallas as pl
from jax.experimental.pallas import tpu as pltpu
from jax.experimental.pallas import tpu_sc as plsc

C = 10
N = 16384
D = 64
L = 16
NW = 32
RPW = N // NW
NCH = D // L
NG = RPW // L
CD = C * D
CL = C * L


def _sqrt16(a):
    i = lax.bitcast_convert_type(a, jnp.int32)
    z = lax.bitcast_convert_type(jnp.int32(0x5F3759DF) - (i >> 1), jnp.float32)
    for _ in range(3):
        z = z * (1.5 - (0.5 * a) * z * z)
    return a * z


def _sc_body(
    pred_hbm, tgt_hbm, centp_hbm,
    partA_hbm, partB_hbm,
    pred_v, tgt_v, centp_v, acc_v, accvl_v,
):
    wid = lax.axis_index("s") * 2 + lax.axis_index("c")
    base = wid * RPW

    pltpu.sync_copy(pred_hbm.at[pl.ds(base * D, RPW * D)], pred_v)
    pltpu.sync_copy(tgt_hbm.at[pl.ds(base, RPW)], tgt_v)
    pltpu.sync_copy(centp_hbm, centp_v)

    zv = jnp.zeros((L,), jnp.float32)
    for q in range(CD // L):
        acc_v[pl.ds(q * L, L)] = zv
    for q in range(CL // L):
        accvl_v[pl.ds(q * L, L)] = zv

    lane_iota = lax.iota(jnp.int32, L)

    def grp_body(g2, carry):
        for half in range(2):
            g = g2 * 2 + half
            tvec = tgt_v[pl.ds(g * L, L)]
            svec = jnp.zeros((L,), jnp.float32)
            for lane in range(L):
                cls = tvec[lane]
                rb = (g * L + lane) * D
                cb = cls * D
                s2 = jnp.zeros((L,), jnp.float32)
                for k in range(NCH):
                    x = pred_v[pl.ds(rb + k * L, L)]
                    diff = centp_v[pl.ds(cb + k * L, L)] - x
                    s2 = s2 + diff * diff
                    plsc.addupdate(acc_v.at[pl.ds(cb + k * L, L)], x)
                svec = jnp.where(lane_iota == lane, jnp.sum(s2), svec)
            nrm = _sqrt16(svec)
            plsc.addupdate_scatter(accvl_v, [tvec * L + lane_iota], nrm)
        return carry

    lax.fori_loop(0, NG // 2, grp_body, 0)

    pltpu.sync_copy(acc_v, partA_hbm.at[pl.ds(wid * CD, CD)])
    pltpu.sync_copy(accvl_v, partB_hbm.at[pl.ds(wid * CL, CL)])


def _tc_tail_body(partA_ref, partB_ref, cent_ref, count_ref, dist_ref, w_ref,
                  out_ref):
    q = 1.0 / count_ref[...][:, 0]
    c2 = cent_ref[...] + q[:, None] * jnp.sum(partA_ref[...], axis=0)
    nrmsum = q * jnp.sum(partB_ref[...], axis=(0, 2))
    s = jnp.sqrt(dist_ref[...][:, 0] + nrmsum) * q
    gram = lax.dot_general(c2, c2, (((1,), (1,)), ((), ())),
                           preferred_element_type=jnp.float32)
    cs = jnp.sum(c2 * c2, axis=1)
    d2 = cs[:, None] + cs[None, :] - 2.0 * gram
    eye = (lax.broadcasted_iota(jnp.int32, (C, C), 0)
           == lax.broadcasted_iota(jnp.int32, (C, C), 1))
    m = jnp.sqrt(jnp.where(eye, 1.0, jnp.maximum(d2, 0.0)))
    pair = w_ref[...] * (s[:, None] + s[None, :]) / m
    total = jnp.sum(jnp.where(eye, 0.0, pair))
    loss = total / C * (C - 1) + jnp.sum(jnp.abs(c2)) / 1000000.0
    out_ref[...] = jnp.full((1, 1), 0.0) + loss


@jax.jit
def _db_loss(pred, tgt, centp, cent2d, count2d, dist2d, w2d):
    mesh = plsc.VectorSubcoreMesh(core_axis_name="c", subcore_axis_name="s")
    params = pltpu.CompilerParams(needs_layout_passes=False)

    sc = functools.partial(
        pl.kernel,
        out_type=[
            jax.ShapeDtypeStruct((NW * CD,), jnp.float32),
            jax.ShapeDtypeStruct((NW * CL,), jnp.float32),
        ],
        mesh=mesh,
        compiler_params=params,
        scratch_types=[
            pltpu.VMEM((RPW * D,), jnp.float32),
            pltpu.VMEM((RPW,), jnp.int32),
            pltpu.VMEM((CD,), jnp.float32),
            pltpu.VMEM((CD,), jnp.float32),
            pltpu.VMEM((CL,), jnp.float32),
        ],
    )(_sc_body)
    partA, partB = sc(pred, tgt, centp)

    out = pl.pallas_call(
        _tc_tail_body,
        in_specs=[
            pl.BlockSpec((NW, C, D), lambda: (0, 0, 0)),
            pl.BlockSpec((NW, C, L), lambda: (0, 0, 0)),
            pl.BlockSpec((C, D), lambda: (0, 0)),
            pl.BlockSpec((C, 1), lambda: (0, 0)),
            pl.BlockSpec((C, 1), lambda: (0, 0)),
            pl.BlockSpec((C, C), lambda: (0, 0)),
        ],
        out_specs=pl.BlockSpec((1, 1), lambda: (0, 0)),
        out_shape=jax.ShapeDtypeStruct((1, 1), jnp.float32),
    )(partA.reshape(NW, C, D), partB.reshape(NW, C, L), cent2d,
      count2d, dist2d, w2d)
    return out


def kernel(predicted, centroids, count, distances, class_weights_matrix, target, epoch):
    centp = (centroids * count).reshape(CD)
    out = _db_loss(predicted.reshape(N * D), target.astype(jnp.int32),
                   centp, centroids, count, distances, class_weights_matrix)
    return out.reshape(1)
